# Initial kernel scaffold; baseline (speedup 1.0000x reference)
import jax
import jax.numpy as jnp
from jax.experimental import pallas as pl


def kernel(x, edge_index, batch, W1, b1, W2, b2):
    raise NotImplementedError("write your pallas kernel here")


# speedup vs baseline: 43.0661x; 43.0661x over previous
import functools

import jax
import jax.numpy as jnp
from jax import lax
from jax.experimental import pallas as pl
from jax.experimental.pallas import tpu as pltpu
from jax.experimental.pallas import tpu_sc as plsc

N = 10000
NP = 10240
E = 320000
G = 128
F1 = 16
NC, NS = 2, 16
NT = NC * NS
EPT = E // NT
RPS = NP // NS
CH = 80
NCH = EPT // CH

_MESH = plsc.VectorSubcoreMesh(
    core_axis_name="c", subcore_axis_name="s", num_cores=NC, num_subcores=NS
)


def _zero_1d(ref, nwords):
    z16 = jnp.zeros((16,), jnp.float32)

    def body(i, _):
        ref[pl.ds(i * 16, 16)] = z16
        return 0

    lax.fori_loop(0, nwords // 16, body, 0)


def _fill_identity(ref, nwords):
    i16 = lax.broadcasted_iota(jnp.int32, (16,), 0)

    def body(i, _):
        ref[pl.ds(i * 16, 16)] = i * 16 + i16
        return 0

    lax.fori_loop(0, nwords // 16, body, 0)


@functools.partial(
    pl.kernel,
    out_type=jax.ShapeDtypeStruct((NC, NP), jnp.float32),
    mesh=_MESH,
    scratch_types=[
        pltpu.VMEM((EPT,), jnp.int32),
        pltpu.VMEM((NP,), jnp.float32),
        pltpu.VMEM((NP,), jnp.int32),
        pltpu.VMEM_SHARED((NP,), jnp.float32),
    ],
    compiler_params=pltpu.CompilerParams(needs_layout_passes=False),
)
def _sc_deg(dst_hbm, out_hbm, dstbuf, degp, idn, spdeg):
    c = lax.axis_index("c")
    s = lax.axis_index("s")
    t = c * NS + s

    _zero_1d(degp, NP)
    _fill_identity(idn, NP)
    pltpu.sync_copy(degp.at[pl.ds(s * RPS, RPS)], spdeg.at[pl.ds(s * RPS, RPS)])
    plsc.subcore_barrier()

    pltpu.sync_copy(dst_hbm.at[pl.ds(t * EPT, EPT)], dstbuf)
    ones = jnp.ones((16,), jnp.float32)

    def body(i, _):
        idx = dstbuf[pl.ds(i * 16, 16)]
        plsc.addupdate_scatter(degp, [idx], ones)
        return 0

    lax.fori_loop(0, EPT // 16, body, 0)

    pltpu.sync_copy(degp, spdeg.at[idn], add=True)
    plsc.subcore_barrier()
    pltpu.sync_copy(spdeg.at[pl.ds(s * RPS, RPS)], out_hbm.at[c, pl.ds(s * RPS, RPS)])


@functools.partial(
    pl.kernel,
    out_type=jax.ShapeDtypeStruct((NC, NP, F1), jnp.float32),
    mesh=_MESH,
    scratch_types=[
        pltpu.VMEM((EPT,), jnp.int32),
        pltpu.VMEM((EPT,), jnp.int32),
        pltpu.VMEM((CH,), jnp.int32),
        pltpu.VMEM((CH,), jnp.int32),
        pltpu.VMEM((CH, F1), jnp.float32),
        pltpu.VMEM((RPS, F1), jnp.float32),
        pltpu.VMEM_SHARED((NP, F1), jnp.float32),
        pltpu.SemaphoreType.DMA,
    ],
    compiler_params=pltpu.CompilerParams(use_tc_tiling_on_sc=False),
)
def _sc_agg(xs_hbm, src_hbm, dst_hbm, out_hbm,
            srcbuf, dstbuf, idxg, idxs, rows, zblk, spagg, sem):
    c = lax.axis_index("c")
    s = lax.axis_index("s")
    t = c * NS + s

    z16 = jnp.zeros((16,), jnp.float32)

    def zbody(i, _):
        zblk[i, :] = z16
        return 0

    lax.fori_loop(0, RPS, zbody, 0)
    pltpu.sync_copy(zblk, spagg.at[pl.ds(s * RPS, RPS)])
    plsc.subcore_barrier()

    pltpu.sync_copy(src_hbm.at[pl.ds(t * EPT, EPT)], srcbuf)
    pltpu.sync_copy(dst_hbm.at[pl.ds(t * EPT, EPT)], dstbuf)

    def body(k, _):
        def cp(j, _):
            idxg[pl.ds(j * 16, 16)] = srcbuf[pl.ds(k * CH + j * 16, 16)]
            idxs[pl.ds(j * 16, 16)] = dstbuf[pl.ds(k * CH + j * 16, 16)]
            return 0

        lax.fori_loop(0, CH // 16, cp, 0)
        pltpu.async_copy(xs_hbm.at[idxg], rows, sem).wait()
        pltpu.sync_copy(rows, spagg.at[idxs], add=True)
        return 0

    lax.fori_loop(0, NCH, body, 0)

    plsc.subcore_barrier()
    pltpu.sync_copy(spagg.at[pl.ds(s * RPS, RPS)], out_hbm.at[c, pl.ds(s * RPS, RPS)])


@functools.partial(
    pl.kernel,
    out_type=jax.ShapeDtypeStruct((NC, NP), jnp.float32),
    mesh=_MESH,
    scratch_types=[
        pltpu.VMEM((NP,), jnp.float32),
        pltpu.VMEM((EPT,), jnp.int32),
        pltpu.VMEM((EPT,), jnp.int32),
        pltpu.VMEM((NP,), jnp.float32),
        pltpu.VMEM((NP,), jnp.int32),
        pltpu.VMEM_SHARED((NP,), jnp.float32),
    ],
    compiler_params=pltpu.CompilerParams(needs_layout_passes=False),
)
def _sc_z(u_hbm, src_hbm, dst_hbm, out_hbm, ubuf, srcbuf, dstbuf, zp, idn, spz):
    c = lax.axis_index("c")
    s = lax.axis_index("s")
    t = c * NS + s

    _zero_1d(zp, NP)
    _fill_identity(idn, NP)
    pltpu.sync_copy(zp.at[pl.ds(s * RPS, RPS)], spz.at[pl.ds(s * RPS, RPS)])
    plsc.subcore_barrier()

    pltpu.sync_copy(u_hbm, ubuf.at[pl.ds(0, N)])
    pltpu.sync_copy(src_hbm.at[pl.ds(t * EPT, EPT)], srcbuf)
    pltpu.sync_copy(dst_hbm.at[pl.ds(t * EPT, EPT)], dstbuf)

    def body(i, _):
        si = srcbuf[pl.ds(i * 16, 16)]
        di = dstbuf[pl.ds(i * 16, 16)]
        vals = plsc.load_gather(ubuf, [si])
        plsc.addupdate_scatter(zp, [di], vals)
        return 0

    lax.fori_loop(0, EPT // 16, body, 0)

    pltpu.sync_copy(zp, spz.at[idn], add=True)
    plsc.subcore_barrier()
    pltpu.sync_copy(spz.at[pl.ds(s * RPS, RPS)], out_hbm.at[c, pl.ds(s * RPS, RPS)])


def _tc1_body(x_ref, w1_ref, dinv_ref, xs_ref):
    xw = jnp.dot(x_ref[...], w1_ref[...], preferred_element_type=jnp.float32)
    xs_ref[...] = xw * dinv_ref[...]


def _tc1(x, W1, dinv):
    return pl.pallas_call(
        _tc1_body,
        out_shape=jax.ShapeDtypeStruct((N, F1), jnp.float32),
    )(x, W1, dinv)


def _tc2_body(agg2_ref, xs_ref, dinv_ref, b1_ref, w2_ref, u_ref):
    agg = agg2_ref[0] + agg2_ref[1]
    dinv = dinv_ref[...]
    h = jnp.maximum(dinv * (agg + xs_ref[...]) + b1_ref[...], 0.0)
    hw = jnp.dot(h, w2_ref[...], preferred_element_type=jnp.float32)
    u_ref[...] = dinv * hw


def _tc2(agg2, xs, dinv, b1, W2):
    return pl.pallas_call(
        _tc2_body,
        out_shape=jax.ShapeDtypeStruct((N, 1), jnp.float32),
    )(agg2, xs, dinv, b1, W2)


def _tc3_body(z_ref, u_ref, dinv_ref, batch_ref, b2_ref, out_ref):
    v = dinv_ref[...] * (z_ref[...] + u_ref[...]) + b2_ref[...]
    gid = lax.broadcasted_iota(jnp.int32, (N, G), 1)
    mask = batch_ref[...] == gid
    w = jnp.where(mask, jnp.broadcast_to(v, (N, G)), 0.0)
    out_ref[...] = jnp.sum(w, axis=0, keepdims=True)


def _tc3(z, u, dinv, batch2d, b2):
    return pl.pallas_call(
        _tc3_body,
        out_shape=jax.ShapeDtypeStruct((1, G), jnp.float32),
    )(z, u, dinv, batch2d, b2)


def kernel(x, edge_index, batch, W1, b1, W2, b2):
    ei = edge_index.astype(jnp.int32)
    src = ei[0]
    dst = ei[1]

    deg2 = _sc_deg(dst)
    deg = deg2[0, :N] + deg2[1, :N]
    dinv = lax.rsqrt(deg + 1.0)[:, None]

    xs = _tc1(x, W1, dinv)
    agg2 = _sc_agg(xs, src, dst)[:, :N]
    u = _tc2(agg2, xs, dinv, b1.reshape(1, F1), W2)

    z2 = _sc_z(u.reshape(N), src, dst)
    z = (z2[0, :N] + z2[1, :N])[:, None]

    out = _tc3(z, u, dinv, batch.astype(jnp.int32)[:, None], b2.reshape(1, 1))
    return out.reshape(G, 1)


# speedup vs baseline: 59.3261x; 1.3776x over previous
import functools

import jax
import jax.numpy as jnp
from jax import lax
from jax.experimental import pallas as pl
from jax.experimental.pallas import tpu as pltpu
from jax.experimental.pallas import tpu_sc as plsc

N = 10000
NP = 10240
E = 320000
G = 128
F1 = 16
NC, NS = 2, 16
NT = NC * NS
EPT = E // NT
RPS = NP // NS
CH = 80
NCH = EPT // CH
NBUF = 4

_MESH = plsc.VectorSubcoreMesh(
    core_axis_name="c", subcore_axis_name="s", num_cores=NC, num_subcores=NS
)


def _zero_1d(ref, nwords):
    z16 = jnp.zeros((16,), jnp.float32)

    def body(i, _):
        ref[pl.ds(i * 16, 16)] = z16
        return 0

    lax.fori_loop(0, nwords // 16, body, 0)


def _fill_identity(ref, nwords):
    i16 = lax.broadcasted_iota(jnp.int32, (16,), 0)

    def body(i, _):
        ref[pl.ds(i * 16, 16)] = i * 16 + i16
        return 0

    lax.fori_loop(0, nwords // 16, body, 0)


@functools.partial(
    pl.kernel,
    out_type=jax.ShapeDtypeStruct((NC, NP), jnp.float32),
    mesh=_MESH,
    scratch_types=[
        pltpu.VMEM((EPT,), jnp.int32),
        pltpu.VMEM((NP,), jnp.float32),
        pltpu.VMEM((NP,), jnp.int32),
        pltpu.VMEM_SHARED((NP,), jnp.float32),
    ],
    compiler_params=pltpu.CompilerParams(needs_layout_passes=False),
)
def _sc_deg(dst_hbm, out_hbm, dstbuf, degp, idn, spdeg):
    c = lax.axis_index("c")
    s = lax.axis_index("s")
    t = c * NS + s

    _zero_1d(degp, NP)
    _fill_identity(idn, NP)
    pltpu.sync_copy(degp.at[pl.ds(s * RPS, RPS)], spdeg.at[pl.ds(s * RPS, RPS)])
    plsc.subcore_barrier()

    pltpu.sync_copy(dst_hbm.at[pl.ds(t * EPT, EPT)], dstbuf)
    ones = jnp.ones((16,), jnp.float32)

    def body(i, _):
        idx = dstbuf[pl.ds(i * 16, 16)]
        plsc.addupdate_scatter(degp, [idx], ones)
        return 0

    lax.fori_loop(0, EPT // 16, body, 0)

    pltpu.sync_copy(degp, spdeg.at[idn], add=True)
    plsc.subcore_barrier()
    pltpu.sync_copy(spdeg.at[pl.ds(s * RPS, RPS)], out_hbm.at[c, pl.ds(s * RPS, RPS)])


@functools.partial(
    pl.kernel,
    out_type=jax.ShapeDtypeStruct((NC, NP, F1), jnp.float32),
    mesh=_MESH,
    scratch_types=[
        pltpu.VMEM((EPT,), jnp.int32),
        pltpu.VMEM((EPT,), jnp.int32),
        pltpu.VMEM((NBUF, CH), jnp.int32),
        pltpu.VMEM((NBUF, CH), jnp.int32),
        pltpu.VMEM((NBUF, CH, F1), jnp.float32),
        pltpu.VMEM((RPS, F1), jnp.float32),
        pltpu.VMEM_SHARED((NP, F1), jnp.float32),
        pltpu.SemaphoreType.DMA,
        pltpu.SemaphoreType.DMA,
        pltpu.SemaphoreType.DMA,
        pltpu.SemaphoreType.DMA,
    ],
    compiler_params=pltpu.CompilerParams(use_tc_tiling_on_sc=False),
)
def _sc_agg(xs_hbm, src_hbm, dst_hbm, out_hbm,
            srcbuf, dstbuf, idxg, idxs, rows, zblk, spagg, sm0, sm1, sm2, sm3):
    sems = (sm0, sm1, sm2, sm3)
    c = lax.axis_index("c")
    s = lax.axis_index("s")
    t = c * NS + s

    z16 = jnp.zeros((16,), jnp.float32)

    def zbody(i, _):
        zblk[i, :] = z16
        return 0

    lax.fori_loop(0, RPS, zbody, 0)
    pltpu.sync_copy(zblk, spagg.at[pl.ds(s * RPS, RPS)])
    plsc.subcore_barrier()

    pltpu.sync_copy(src_hbm.at[pl.ds(t * EPT, EPT)], srcbuf)
    pltpu.sync_copy(dst_hbm.at[pl.ds(t * EPT, EPT)], dstbuf)

    def stage(k, b):
        def cp(j, _):
            idxg[b, pl.ds(j * 16, 16)] = srcbuf[pl.ds(k * CH + j * 16, 16)]
            idxs[b, pl.ds(j * 16, 16)] = dstbuf[pl.ds(k * CH + j * 16, 16)]
            return 0

        lax.fori_loop(0, CH // 16, cp, 0)
        pltpu.async_copy(xs_hbm.at[idxg.at[b]], rows.at[b], sems[b])

    for b in range(NBUF - 1):
        stage(b, b)

    def body(i, _):
        for b in range(NBUF):
            k = i * NBUF + b
            pltpu.make_async_copy(xs_hbm.at[idxg.at[b]], rows.at[b], sems[b]).wait()
            pltpu.sync_copy(rows.at[b], spagg.at[idxs.at[b]], add=True)

            @pl.when(k + NBUF - 1 < NCH)
            def _():
                stage(k + NBUF - 1, (b + NBUF - 1) % NBUF)

        return 0

    lax.fori_loop(0, NCH // NBUF, body, 0)
    for k in range(NCH - NCH % NBUF, NCH):
        b = k % NBUF
        pltpu.make_async_copy(xs_hbm.at[idxg.at[b]], rows.at[b], sems[b]).wait()
        pltpu.sync_copy(rows.at[b], spagg.at[idxs.at[b]], add=True)

    plsc.subcore_barrier()
    pltpu.sync_copy(spagg.at[pl.ds(s * RPS, RPS)], out_hbm.at[c, pl.ds(s * RPS, RPS)])


@functools.partial(
    pl.kernel,
    out_type=jax.ShapeDtypeStruct((NC, NP), jnp.float32),
    mesh=_MESH,
    scratch_types=[
        pltpu.VMEM((NP,), jnp.float32),
        pltpu.VMEM((EPT,), jnp.int32),
        pltpu.VMEM((EPT,), jnp.int32),
        pltpu.VMEM((NP,), jnp.float32),
        pltpu.VMEM((NP,), jnp.int32),
        pltpu.VMEM_SHARED((NP,), jnp.float32),
    ],
    compiler_params=pltpu.CompilerParams(needs_layout_passes=False),
)
def _sc_z(u_hbm, src_hbm, dst_hbm, out_hbm, ubuf, srcbuf, dstbuf, zp, idn, spz):
    c = lax.axis_index("c")
    s = lax.axis_index("s")
    t = c * NS + s

    _zero_1d(zp, NP)
    _fill_identity(idn, NP)
    pltpu.sync_copy(zp.at[pl.ds(s * RPS, RPS)], spz.at[pl.ds(s * RPS, RPS)])
    plsc.subcore_barrier()

    pltpu.sync_copy(u_hbm, ubuf.at[pl.ds(0, N)])
    pltpu.sync_copy(src_hbm.at[pl.ds(t * EPT, EPT)], srcbuf)
    pltpu.sync_copy(dst_hbm.at[pl.ds(t * EPT, EPT)], dstbuf)

    def body(i, _):
        si = srcbuf[pl.ds(i * 16, 16)]
        di = dstbuf[pl.ds(i * 16, 16)]
        vals = plsc.load_gather(ubuf, [si])
        plsc.addupdate_scatter(zp, [di], vals)
        return 0

    lax.fori_loop(0, EPT // 16, body, 0)

    pltpu.sync_copy(zp, spz.at[idn], add=True)
    plsc.subcore_barrier()
    pltpu.sync_copy(spz.at[pl.ds(s * RPS, RPS)], out_hbm.at[c, pl.ds(s * RPS, RPS)])


def _tc1_body(x_ref, w1_ref, dinv_ref, xs_ref):
    xw = jnp.dot(x_ref[...], w1_ref[...], preferred_element_type=jnp.float32)
    xs_ref[...] = xw * dinv_ref[...]


def _tc1(x, W1, dinv):
    return pl.pallas_call(
        _tc1_body,
        out_shape=jax.ShapeDtypeStruct((N, F1), jnp.float32),
    )(x, W1, dinv)


def _tc2_body(agg2_ref, xs_ref, dinv_ref, b1_ref, w2_ref, u_ref):
    agg = agg2_ref[0] + agg2_ref[1]
    dinv = dinv_ref[...]
    h = jnp.maximum(dinv * (agg + xs_ref[...]) + b1_ref[...], 0.0)
    hw = jnp.dot(h, w2_ref[...], preferred_element_type=jnp.float32)
    u_ref[...] = dinv * hw


def _tc2(agg2, xs, dinv, b1, W2):
    return pl.pallas_call(
        _tc2_body,
        out_shape=jax.ShapeDtypeStruct((N, 1), jnp.float32),
    )(agg2, xs, dinv, b1, W2)


def _tc3_body(z_ref, u_ref, dinv_ref, batch_ref, b2_ref, out_ref):
    v = dinv_ref[...] * (z_ref[...] + u_ref[...]) + b2_ref[...]
    gid = lax.broadcasted_iota(jnp.int32, (N, G), 1)
    mask = batch_ref[...] == gid
    w = jnp.where(mask, jnp.broadcast_to(v, (N, G)), 0.0)
    out_ref[...] = jnp.sum(w, axis=0, keepdims=True)


def _tc3(z, u, dinv, batch2d, b2):
    return pl.pallas_call(
        _tc3_body,
        out_shape=jax.ShapeDtypeStruct((1, G), jnp.float32),
    )(z, u, dinv, batch2d, b2)


def kernel(x, edge_index, batch, W1, b1, W2, b2):
    ei = edge_index.astype(jnp.int32)
    src = ei[0]
    dst = ei[1]

    deg2 = _sc_deg(dst)
    deg = deg2[0, :N] + deg2[1, :N]
    dinv = lax.rsqrt(deg + 1.0)[:, None]

    xs = _tc1(x, W1, dinv)
    agg2 = _sc_agg(xs, src, dst)[:, :N]
    u = _tc2(agg2, xs, dinv, b1.reshape(1, F1), W2)

    z2 = _sc_z(u.reshape(N), src, dst)
    z = (z2[0, :N] + z2[1, :N])[:, None]

    out = _tc3(z, u, dinv, batch.astype(jnp.int32)[:, None], b2.reshape(1, 1))
    return out.reshape(G, 1)


# speedup vs baseline: 71.8572x; 1.2112x over previous
import functools

import jax
import jax.numpy as jnp
from jax import lax
from jax.experimental import pallas as pl
from jax.experimental.pallas import tpu as pltpu
from jax.experimental.pallas import tpu_sc as plsc

N = 10000
NP = 10240
E = 320000
G = 128
F1 = 16
NC, NS = 2, 16
NT = NC * NS
EPT = E // NT
RPS = NP // NS
CH = 80
NCH = EPT // CH
NBUF = 4

_MESH = plsc.VectorSubcoreMesh(
    core_axis_name="c", subcore_axis_name="s", num_cores=NC, num_subcores=NS
)


def _zero_1d(ref, nwords):
    z16 = jnp.zeros((16,), jnp.float32)

    def body(i, _):
        ref[pl.ds(i * 16, 16)] = z16
        return 0

    lax.fori_loop(0, nwords // 16, body, 0)


def _fill_identity(ref, nwords):
    i16 = lax.broadcasted_iota(jnp.int32, (16,), 0)

    def body(i, _):
        ref[pl.ds(i * 16, 16)] = i * 16 + i16
        return 0

    lax.fori_loop(0, nwords // 16, body, 0)


@functools.partial(
    pl.kernel,
    out_type=jax.ShapeDtypeStruct((NC, NP), jnp.float32),
    mesh=_MESH,
    scratch_types=[
        pltpu.VMEM((EPT,), jnp.int32),
        pltpu.VMEM((NP,), jnp.float32),
        pltpu.VMEM((NP,), jnp.int32),
        pltpu.VMEM_SHARED((NP,), jnp.float32),
    ],
    compiler_params=pltpu.CompilerParams(needs_layout_passes=False),
)
def _sc_deg(dst_hbm, out_hbm, dstbuf, degp, idn, spdeg):
    c = lax.axis_index("c")
    s = lax.axis_index("s")
    t = c * NS + s

    _zero_1d(degp, NP)
    _fill_identity(idn, NP)
    pltpu.sync_copy(degp.at[pl.ds(s * RPS, RPS)], spdeg.at[pl.ds(s * RPS, RPS)])
    plsc.subcore_barrier()

    pltpu.sync_copy(dst_hbm.at[pl.ds(t * EPT, EPT)], dstbuf)
    ones = jnp.ones((16,), jnp.float32)

    def body(i, _):
        idx = dstbuf[pl.ds(i * 16, 16)]
        plsc.addupdate_scatter(degp, [idx], ones)
        return 0

    lax.fori_loop(0, EPT // 16, body, 0)

    pltpu.sync_copy(degp, spdeg.at[idn], add=True)
    plsc.subcore_barrier()
    pltpu.sync_copy(spdeg.at[pl.ds(s * RPS, RPS)], out_hbm.at[c, pl.ds(s * RPS, RPS)])


@functools.partial(
    pl.kernel,
    out_type=jax.ShapeDtypeStruct((NC, NP, F1), jnp.float32),
    mesh=_MESH,
    scratch_types=[
        pltpu.VMEM((EPT,), jnp.int32),
        pltpu.VMEM((EPT,), jnp.int32),
        pltpu.VMEM((NBUF, CH), jnp.int32),
        pltpu.VMEM((NBUF, CH), jnp.int32),
        pltpu.VMEM((NBUF, CH, F1), jnp.float32),
        pltpu.VMEM((RPS, F1), jnp.float32),
        pltpu.VMEM_SHARED((NP, F1), jnp.float32),
        pltpu.SemaphoreType.DMA,
        pltpu.SemaphoreType.DMA,
        pltpu.SemaphoreType.DMA,
        pltpu.SemaphoreType.DMA,
    ],
    compiler_params=pltpu.CompilerParams(use_tc_tiling_on_sc=False),
)
def _sc_agg(xs_hbm, src_hbm, dst_hbm, out_hbm,
            srcbuf, dstbuf, idxg, idxs, rows, zblk, spagg, sm0, sm1, sm2, sm3):
    sems = (sm0, sm1, sm2, sm3)
    c = lax.axis_index("c")
    s = lax.axis_index("s")
    t = c * NS + s

    z16 = jnp.zeros((16,), jnp.float32)

    def zbody(i, _):
        zblk[i, :] = z16
        return 0

    lax.fori_loop(0, RPS, zbody, 0)
    pltpu.sync_copy(zblk, spagg.at[pl.ds(s * RPS, RPS)])
    plsc.subcore_barrier()

    pltpu.sync_copy(src_hbm.at[pl.ds(t * EPT, EPT)], srcbuf)
    pltpu.sync_copy(dst_hbm.at[pl.ds(t * EPT, EPT)], dstbuf)

    def stage(k, b):
        def cp(j, _):
            idxg[b, pl.ds(j * 16, 16)] = srcbuf[pl.ds(k * CH + j * 16, 16)]
            idxs[b, pl.ds(j * 16, 16)] = dstbuf[pl.ds(k * CH + j * 16, 16)]
            return 0

        lax.fori_loop(0, CH // 16, cp, 0)
        pltpu.async_copy(xs_hbm.at[idxg.at[b]], rows.at[b], sems[b])

    for b in range(NBUF - 1):
        stage(b, b)

    def body(i, _):
        for b in range(NBUF):
            k = i * NBUF + b
            pltpu.make_async_copy(xs_hbm.at[idxg.at[b]], rows.at[b], sems[b]).wait()
            pltpu.sync_copy(rows.at[b], spagg.at[idxs.at[b]], add=True)

            @pl.when(k + NBUF - 1 < NCH)
            def _():
                stage(k + NBUF - 1, (b + NBUF - 1) % NBUF)

        return 0

    lax.fori_loop(0, NCH // NBUF, body, 0)
    for k in range(NCH - NCH % NBUF, NCH):
        b = k % NBUF
        pltpu.make_async_copy(xs_hbm.at[idxg.at[b]], rows.at[b], sems[b]).wait()
        pltpu.sync_copy(rows.at[b], spagg.at[idxs.at[b]], add=True)

    plsc.subcore_barrier()
    pltpu.sync_copy(spagg.at[pl.ds(s * RPS, RPS)], out_hbm.at[c, pl.ds(s * RPS, RPS)])


@functools.partial(
    pl.kernel,
    out_type=jax.ShapeDtypeStruct((NC, G), jnp.float32),
    mesh=_MESH,
    scratch_types=[
        pltpu.VMEM((RPS, F1), jnp.float32),
        pltpu.VMEM((RPS, F1), jnp.float32),
        pltpu.VMEM((RPS, F1), jnp.float32),
        pltpu.VMEM((RPS,), jnp.float32),
        pltpu.VMEM((RPS,), jnp.int32),
        pltpu.VMEM((RPS,), jnp.float32),
        pltpu.VMEM((RPS,), jnp.float32),
        pltpu.VMEM((NP,), jnp.float32),
        pltpu.VMEM((EPT,), jnp.int32),
        pltpu.VMEM((EPT,), jnp.int32),
        pltpu.VMEM((NP,), jnp.float32),
        pltpu.VMEM((NP,), jnp.int32),
        pltpu.VMEM((G,), jnp.float32),
        pltpu.VMEM((G,), jnp.int32),
        pltpu.VMEM((F1, 16), jnp.float32),
        pltpu.VMEM((F1, 16), jnp.float32),
        pltpu.VMEM((F1,), jnp.float32),
        pltpu.VMEM_SHARED((NP,), jnp.float32),
        pltpu.VMEM_SHARED((NP,), jnp.float32),
        pltpu.VMEM_SHARED((G,), jnp.float32),
    ],
    compiler_params=pltpu.CompilerParams(
        needs_layout_passes=False, use_tc_tiling_on_sc=False
    ),
)
def _sc_zu(agg2_hbm, xs_hbm, dinv_hbm, batch_hbm, src_hbm, dst_hbm,
           w2_hbm, b1_hbm, b2_hbm, out_hbm,
           a0, a1, xsl, dvl, btl, uloc, zl, ubuf, srcbuf, dstbuf,
           zp, idn, bins, binidn, w2m, b1m, b2b, spu, spz, spbins):
    c = lax.axis_index("c")
    s = lax.axis_index("s")
    t = c * NS + s
    base = s * RPS

    _zero_1d(zp, NP)
    _fill_identity(idn, NP)
    _zero_1d(bins, G)
    _fill_identity(binidn, G)
    pltpu.sync_copy(zp.at[pl.ds(base, RPS)], spz.at[pl.ds(base, RPS)])

    @pl.when(s == 0)
    def _():
        pltpu.sync_copy(bins, spbins)

    pltpu.sync_copy(agg2_hbm.at[0, pl.ds(base, RPS)], a0)
    pltpu.sync_copy(agg2_hbm.at[1, pl.ds(base, RPS)], a1)
    pltpu.sync_copy(xs_hbm.at[pl.ds(base, RPS)], xsl)
    pltpu.sync_copy(dinv_hbm.at[pl.ds(base, RPS)], dvl)
    pltpu.sync_copy(batch_hbm.at[pl.ds(base, RPS)], btl)
    pltpu.sync_copy(w2_hbm, w2m)
    pltpu.sync_copy(b1_hbm, b1m)
    pltpu.sync_copy(b2_hbm, b2b)
    pltpu.sync_copy(src_hbm.at[pl.ds(t * EPT, EPT)], srcbuf)
    pltpu.sync_copy(dst_hbm.at[pl.ds(t * EPT, EPT)], dstbuf)

    i16a = lax.broadcasted_iota(jnp.int32, (16,), 0)
    b1rows = [b1m[f, :] for f in range(F1)]
    w2rows = [w2m[f, :] for f in range(F1)]

    def ub(i, _):
        rows = i * 16 + i16a
        d16 = dvl[pl.ds(i * 16, 16)]
        acc = jnp.zeros((16,), jnp.float32)
        for f in range(F1):
            cf = jnp.full((16,), f, jnp.int32)
            g = (plsc.load_gather(a0, [rows, cf])
                 + plsc.load_gather(a1, [rows, cf])
                 + plsc.load_gather(xsl, [rows, cf]))
            hf = jnp.maximum(g * d16 + b1rows[f], 0.0)
            bits = plsc.bitcast(hf, jnp.int32)
            lsb = jnp.bitwise_and(lax.shift_right_logical(bits, 16), 1)
            rb = jnp.bitwise_and(bits + 0x7FFF + lsb, jnp.int32(-65536))
            acc = acc + plsc.bitcast(rb, jnp.float32) * w2rows[f]
        uloc[pl.ds(i * 16, 16)] = acc * d16
        return 0

    lax.fori_loop(0, RPS // 16, ub, 0)
    pltpu.sync_copy(uloc, spu.at[pl.ds(base, RPS)])
    plsc.subcore_barrier()

    pltpu.sync_copy(spu, ubuf)

    def eb(i, _):
        si = srcbuf[pl.ds(i * 16, 16)]
        di = dstbuf[pl.ds(i * 16, 16)]
        vals = plsc.load_gather(ubuf, [si])
        plsc.addupdate_scatter(zp, [di], vals)
        return 0

    lax.fori_loop(0, EPT // 16, eb, 0)
    pltpu.sync_copy(zp, spz.at[idn], add=True)
    plsc.subcore_barrier()

    pltpu.sync_copy(spz.at[pl.ds(base, RPS)], zl)
    hb2 = b2b[...] * 0.5
    i16 = lax.broadcasted_iota(jnp.int32, (16,), 0)

    def wb(i, _):
        z16 = zl[pl.ds(i * 16, 16)]
        u16 = ubuf[pl.ds(base + i * 16, 16)]
        d16 = dvl[pl.ds(i * 16, 16)]
        b16 = btl[pl.ds(i * 16, 16)]
        row = base + i * 16 + i16
        w16 = d16 * (z16 + 0.5 * u16) + hb2
        w16 = jnp.where(row < N, w16, 0.0)
        plsc.addupdate_scatter(bins, [b16], w16)
        return 0

    lax.fori_loop(0, RPS // 16, wb, 0)
    pltpu.sync_copy(bins, spbins.at[binidn], add=True)
    plsc.subcore_barrier()

    @pl.when(s == 0)
    def _():
        pltpu.sync_copy(spbins, out_hbm.at[c])


def _tc1_body(x_ref, w1_ref, dinv_ref, xs_ref):
    xw = jnp.dot(x_ref[...], w1_ref[...], preferred_element_type=jnp.float32)
    xs_ref[...] = xw * dinv_ref[...]


def _tc1(x, W1, dinv):
    return pl.pallas_call(
        _tc1_body,
        out_shape=jax.ShapeDtypeStruct((NP, F1), jnp.float32),
    )(x, W1, dinv)


def kernel(x, edge_index, batch, W1, b1, W2, b2):
    ei = edge_index.astype(jnp.int32)
    src = ei[0]
    dst = ei[1]
    x_p = jnp.pad(x, ((0, NP - N), (0, 0)))
    batch_p = jnp.pad(batch.astype(jnp.int32), (0, NP - N))

    deg2 = _sc_deg(dst)
    dinv = lax.rsqrt(deg2[0] + deg2[1] + 1.0)

    xs = _tc1(x_p, W1, dinv[:, None])
    agg2 = _sc_agg(xs, src, dst)

    w2r = W2.astype(jnp.bfloat16).astype(jnp.float32).reshape(F1)
    out2 = _sc_zu(agg2, xs, dinv, batch_p, src, dst,
                  jnp.broadcast_to(w2r[:, None], (F1, 16)),
                  jnp.broadcast_to(b1[:, None], (F1, 16)),
                  jnp.broadcast_to(b2, (F1,)))
    return (out2[0] + out2[1]).reshape(G, 1)


# speedup vs baseline: 78.6169x; 1.0941x over previous
import functools

import jax
import jax.numpy as jnp
from jax import lax
from jax.experimental import pallas as pl
from jax.experimental.pallas import tpu as pltpu
from jax.experimental.pallas import tpu_sc as plsc

N = 10000
NP = 10240
E = 320000
G = 128
F1 = 16
NC, NS = 2, 16
NT = NC * NS
EPT = E // NT
RPS = NP // NS
CH = 80
NCH = EPT // CH
NBUF = 4

_MESH = plsc.VectorSubcoreMesh(
    core_axis_name="c", subcore_axis_name="s", num_cores=NC, num_subcores=NS
)


def _zero_1d(ref, nwords):
    z16 = jnp.zeros((16,), jnp.float32)

    def body(i, _):
        ref[pl.ds(i * 16, 16)] = z16
        return 0

    lax.fori_loop(0, nwords // 16, body, 0)


def _fill_identity(ref, nwords):
    i16 = lax.broadcasted_iota(jnp.int32, (16,), 0)

    def body(i, _):
        ref[pl.ds(i * 16, 16)] = i * 16 + i16
        return 0

    lax.fori_loop(0, nwords // 16, body, 0)


@functools.partial(
    pl.kernel,
    out_type=jax.ShapeDtypeStruct((NC, NP), jnp.float32),
    mesh=_MESH,
    scratch_types=[
        pltpu.VMEM((EPT,), jnp.int32),
        pltpu.VMEM((NP,), jnp.float32),
        pltpu.VMEM((NP,), jnp.int32),
        pltpu.VMEM_SHARED((NP,), jnp.float32),
    ],
    compiler_params=pltpu.CompilerParams(needs_layout_passes=False),
)
def _sc_deg(dst_hbm, out_hbm, dstbuf, degp, idn, spdeg):
    c = lax.axis_index("c")
    s = lax.axis_index("s")
    t = c * NS + s

    _zero_1d(degp, NP)
    _fill_identity(idn, NP)
    pltpu.sync_copy(degp.at[pl.ds(s * RPS, RPS)], spdeg.at[pl.ds(s * RPS, RPS)])
    plsc.subcore_barrier()

    pltpu.sync_copy(dst_hbm.at[pl.ds(t * EPT, EPT)], dstbuf)
    ones = jnp.ones((16,), jnp.float32)

    def body(i, _):
        idx = dstbuf[pl.ds(i * 16, 16)]
        plsc.addupdate_scatter(degp, [idx], ones)
        return 0

    lax.fori_loop(0, EPT // 16, body, 0)

    pltpu.sync_copy(degp, spdeg.at[idn], add=True)
    plsc.subcore_barrier()
    pltpu.sync_copy(spdeg.at[pl.ds(s * RPS, RPS)], out_hbm.at[c, pl.ds(s * RPS, RPS)])


@functools.partial(
    pl.kernel,
    out_type=jax.ShapeDtypeStruct((NC, NP, F1), jnp.float32),
    mesh=_MESH,
    scratch_types=[
        pltpu.VMEM((EPT,), jnp.int32),
        pltpu.VMEM((EPT,), jnp.int32),
        pltpu.VMEM((NBUF, CH), jnp.int32),
        pltpu.VMEM((NBUF, CH), jnp.int32),
        pltpu.VMEM((NBUF, CH, F1), jnp.float32),
        pltpu.VMEM((RPS, F1), jnp.float32),
        pltpu.VMEM_SHARED((NP, F1), jnp.float32),
        pltpu.VMEM_SHARED((NP, F1), jnp.float32),
        pltpu.SemaphoreType.DMA,
        pltpu.SemaphoreType.DMA,
        pltpu.SemaphoreType.DMA,
        pltpu.SemaphoreType.DMA,
    ],
    compiler_params=pltpu.CompilerParams(use_tc_tiling_on_sc=False),
)
def _sc_agg(xs_hbm, src_hbm, dst_hbm, out_hbm,
            srcbuf, dstbuf, idxg, idxs, rows, zblk, spagg, spxs, sm0, sm1, sm2, sm3):
    sems = (sm0, sm1, sm2, sm3)
    c = lax.axis_index("c")
    s = lax.axis_index("s")
    t = c * NS + s

    z16 = jnp.zeros((16,), jnp.float32)

    def zbody(i, _):
        zblk[i, :] = z16
        return 0

    lax.fori_loop(0, RPS, zbody, 0)
    pltpu.sync_copy(zblk, spagg.at[pl.ds(s * RPS, RPS)])
    pltpu.sync_copy(xs_hbm.at[pl.ds(s * RPS, RPS)], spxs.at[pl.ds(s * RPS, RPS)])
    plsc.subcore_barrier()

    pltpu.sync_copy(src_hbm.at[pl.ds(t * EPT, EPT)], srcbuf)
    pltpu.sync_copy(dst_hbm.at[pl.ds(t * EPT, EPT)], dstbuf)

    def stage(k, b):
        def cp(j, _):
            idxg[b, pl.ds(j * 16, 16)] = srcbuf[pl.ds(k * CH + j * 16, 16)]
            idxs[b, pl.ds(j * 16, 16)] = dstbuf[pl.ds(k * CH + j * 16, 16)]
            return 0

        lax.fori_loop(0, CH // 16, cp, 0)
        pltpu.async_copy(spxs.at[idxg.at[b]], rows.at[b], sems[b])

    for b in range(NBUF - 1):
        stage(b, b)

    def body(i, _):
        for b in range(NBUF):
            k = i * NBUF + b
            pltpu.make_async_copy(spxs.at[idxg.at[b]], rows.at[b], sems[b]).wait()
            pltpu.sync_copy(rows.at[b], spagg.at[idxs.at[b]], add=True)

            @pl.when(k + NBUF - 1 < NCH)
            def _():
                stage(k + NBUF - 1, (b + NBUF - 1) % NBUF)

        return 0

    lax.fori_loop(0, NCH // NBUF, body, 0)
    for k in range(NCH - NCH % NBUF, NCH):
        b = k % NBUF
        pltpu.make_async_copy(spxs.at[idxg.at[b]], rows.at[b], sems[b]).wait()
        pltpu.sync_copy(rows.at[b], spagg.at[idxs.at[b]], add=True)

    plsc.subcore_barrier()
    pltpu.sync_copy(spagg.at[pl.ds(s * RPS, RPS)], out_hbm.at[c, pl.ds(s * RPS, RPS)])


@functools.partial(
    pl.kernel,
    out_type=jax.ShapeDtypeStruct((NC, G), jnp.float32),
    mesh=_MESH,
    scratch_types=[
        pltpu.VMEM((RPS, F1), jnp.float32),
        pltpu.VMEM((RPS, F1), jnp.float32),
        pltpu.VMEM((RPS, F1), jnp.float32),
        pltpu.VMEM((RPS,), jnp.float32),
        pltpu.VMEM((RPS,), jnp.int32),
        pltpu.VMEM((RPS,), jnp.float32),
        pltpu.VMEM((RPS,), jnp.float32),
        pltpu.VMEM((NP,), jnp.float32),
        pltpu.VMEM((EPT,), jnp.int32),
        pltpu.VMEM((EPT,), jnp.int32),
        pltpu.VMEM((NP,), jnp.float32),
        pltpu.VMEM((NP,), jnp.int32),
        pltpu.VMEM((G,), jnp.float32),
        pltpu.VMEM((G,), jnp.int32),
        pltpu.VMEM((F1, 16), jnp.float32),
        pltpu.VMEM((F1, 16), jnp.float32),
        pltpu.VMEM((F1,), jnp.float32),
        pltpu.VMEM_SHARED((NP,), jnp.float32),
        pltpu.VMEM_SHARED((NP,), jnp.float32),
        pltpu.VMEM_SHARED((G,), jnp.float32),
    ],
    compiler_params=pltpu.CompilerParams(
        needs_layout_passes=False, use_tc_tiling_on_sc=False
    ),
)
def _sc_zu(agg2_hbm, xs_hbm, dinv_hbm, batch_hbm, src_hbm, dst_hbm,
           w2_hbm, b1_hbm, b2_hbm, out_hbm,
           a0, a1, xsl, dvl, btl, uloc, zl, ubuf, srcbuf, dstbuf,
           zp, idn, bins, binidn, w2m, b1m, b2b, spu, spz, spbins):
    c = lax.axis_index("c")
    s = lax.axis_index("s")
    t = c * NS + s
    base = s * RPS

    _zero_1d(zp, NP)
    _fill_identity(idn, NP)
    _zero_1d(bins, G)
    _fill_identity(binidn, G)
    pltpu.sync_copy(zp.at[pl.ds(base, RPS)], spz.at[pl.ds(base, RPS)])

    @pl.when(s == 0)
    def _():
        pltpu.sync_copy(bins, spbins)

    pltpu.sync_copy(agg2_hbm.at[0, pl.ds(base, RPS)], a0)
    pltpu.sync_copy(agg2_hbm.at[1, pl.ds(base, RPS)], a1)
    pltpu.sync_copy(xs_hbm.at[pl.ds(base, RPS)], xsl)
    pltpu.sync_copy(dinv_hbm.at[pl.ds(base, RPS)], dvl)
    pltpu.sync_copy(batch_hbm.at[pl.ds(base, RPS)], btl)
    pltpu.sync_copy(w2_hbm, w2m)
    pltpu.sync_copy(b1_hbm, b1m)
    pltpu.sync_copy(b2_hbm, b2b)
    pltpu.sync_copy(src_hbm.at[pl.ds(t * EPT, EPT)], srcbuf)
    pltpu.sync_copy(dst_hbm.at[pl.ds(t * EPT, EPT)], dstbuf)

    i16a = lax.broadcasted_iota(jnp.int32, (16,), 0)
    b1rows = [b1m[f, :] for f in range(F1)]
    w2rows = [w2m[f, :] for f in range(F1)]

    def ub(i, _):
        rows = i * 16 + i16a
        d16 = dvl[pl.ds(i * 16, 16)]
        acc = jnp.zeros((16,), jnp.float32)
        for f in range(F1):
            cf = jnp.full((16,), f, jnp.int32)
            g = (plsc.load_gather(a0, [rows, cf])
                 + plsc.load_gather(a1, [rows, cf])
                 + plsc.load_gather(xsl, [rows, cf]))
            hf = jnp.maximum(g * d16 + b1rows[f], 0.0)
            bits = plsc.bitcast(hf, jnp.int32)
            lsb = jnp.bitwise_and(lax.shift_right_logical(bits, 16), 1)
            rb = jnp.bitwise_and(bits + 0x7FFF + lsb, jnp.int32(-65536))
            acc = acc + plsc.bitcast(rb, jnp.float32) * w2rows[f]
        uloc[pl.ds(i * 16, 16)] = acc * d16
        return 0

    lax.fori_loop(0, RPS // 16, ub, 0)
    pltpu.sync_copy(uloc, spu.at[pl.ds(base, RPS)])
    plsc.subcore_barrier()

    pltpu.sync_copy(spu, ubuf)

    def eb(i, _):
        si = srcbuf[pl.ds(i * 16, 16)]
        di = dstbuf[pl.ds(i * 16, 16)]
        vals = plsc.load_gather(ubuf, [si])
        plsc.addupdate_scatter(zp, [di], vals)
        return 0

    lax.fori_loop(0, EPT // 16, eb, 0)
    pltpu.sync_copy(zp, spz.at[idn], add=True)
    plsc.subcore_barrier()

    pltpu.sync_copy(spz.at[pl.ds(base, RPS)], zl)
    hb2 = b2b[...] * 0.5
    i16 = lax.broadcasted_iota(jnp.int32, (16,), 0)

    def wb(i, _):
        z16 = zl[pl.ds(i * 16, 16)]
        u16 = ubuf[pl.ds(base + i * 16, 16)]
        d16 = dvl[pl.ds(i * 16, 16)]
        b16 = btl[pl.ds(i * 16, 16)]
        row = base + i * 16 + i16
        w16 = d16 * (z16 + 0.5 * u16) + hb2
        w16 = jnp.where(row < N, w16, 0.0)
        plsc.addupdate_scatter(bins, [b16], w16)
        return 0

    lax.fori_loop(0, RPS // 16, wb, 0)
    pltpu.sync_copy(bins, spbins.at[binidn], add=True)
    plsc.subcore_barrier()

    @pl.when(s == 0)
    def _():
        pltpu.sync_copy(spbins, out_hbm.at[c])


def _tc1_body(x_ref, w1_ref, dinv_ref, xs_ref):
    xw = jnp.dot(x_ref[...], w1_ref[...], preferred_element_type=jnp.float32)
    xs_ref[...] = xw * dinv_ref[...]


def _tc1(x, W1, dinv):
    return pl.pallas_call(
        _tc1_body,
        out_shape=jax.ShapeDtypeStruct((NP, F1), jnp.float32),
    )(x, W1, dinv)


def kernel(x, edge_index, batch, W1, b1, W2, b2):
    ei = edge_index.astype(jnp.int32)
    src = ei[0]
    dst = ei[1]
    x_p = jnp.pad(x, ((0, NP - N), (0, 0)))
    batch_p = jnp.pad(batch.astype(jnp.int32), (0, NP - N))

    deg2 = _sc_deg(dst)
    dinv = lax.rsqrt(deg2[0] + deg2[1] + 1.0)

    xs = _tc1(x_p, W1, dinv[:, None])
    agg2 = _sc_agg(xs, src, dst)

    w2r = W2.astype(jnp.bfloat16).astype(jnp.float32).reshape(F1)
    out2 = _sc_zu(agg2, xs, dinv, batch_p, src, dst,
                  jnp.broadcast_to(w2r[:, None], (F1, 16)),
                  jnp.broadcast_to(b1[:, None], (F1, 16)),
                  jnp.broadcast_to(b2, (F1,)))
    return (out2[0] + out2[1]).reshape(G, 1)
